# baseline (speedup 1.0000x reference)
import functools

import jax
import jax.numpy as jnp
from jax import lax
from jax.experimental import pallas as pl
from jax.experimental.pallas import tpu as pltpu
from jax.experimental.pallas import tpu_sc as plsc

_LANES = 16


def _make_sc_scatter(n_pad: int, v_pad: int):
    info = plsc.get_sparse_core_info()
    nc, ns = info.num_cores, info.num_subcores
    nw = nc * ns
    sl = n_pad // nw

    mesh = plsc.VectorSubcoreMesh(core_axis_name="c", subcore_axis_name="s")

    @functools.partial(
        pl.kernel,
        mesh=mesh,
        out_type=jax.ShapeDtypeStruct((n_pad,), jnp.float32),
        scratch_types=[
            pltpu.VMEM((v_pad,), jnp.int32),
            pltpu.VMEM((v_pad,), jnp.float32),
            pltpu.VMEM((sl,), jnp.float32),
        ],
        compiler_params=pltpu.CompilerParams(needs_layout_passes=False),
    )
    def sc_scatter(idx_hbm, w_hbm, out_hbm, idx_v, w_v, acc_v):
        wid = lax.axis_index("s") * nc + lax.axis_index("c")
        base = wid * sl
        pltpu.sync_copy(idx_hbm, idx_v)
        pltpu.sync_copy(w_hbm, w_v)

        zeros = jnp.zeros((_LANES,), jnp.float32)

        def zero_body(i, carry):
            acc_v[pl.ds(i * _LANES, _LANES)] = zeros
            return carry

        lax.fori_loop(0, sl // _LANES, zero_body, 0)

        def scatter_body(i, carry):
            iv = idx_v[pl.ds(i * _LANES, _LANES)]
            wv = w_v[pl.ds(i * _LANES, _LANES)]
            loc = iv - base
            m = (loc >= 0) & (loc < sl)
            loc = jnp.where(m, loc, 0)
            plsc.store_scatter(acc_v, [loc], wv, mask=m)
            return carry

        lax.fori_loop(0, v_pad // _LANES, scatter_body, 0)

        pltpu.sync_copy(acc_v, out_hbm.at[pl.ds(base, sl)])

    return sc_scatter


def _matvec_body(y_ref, w_ref, b_ref, o_ref):
    o_ref[...] = (
        jnp.sum(y_ref[...] * w_ref[...], axis=1, keepdims=True) + b_ref[0, 0]
    )


def _tc_matvec(y, w2d, bias2d, row_block: int):
    b, n = y.shape
    return pl.pallas_call(
        _matvec_body,
        grid=(b // row_block,),
        in_specs=[
            pl.BlockSpec((row_block, n), lambda i: (i, 0)),
            pl.BlockSpec((1, n), lambda i: (0, 0)),
            pl.BlockSpec(memory_space=pltpu.SMEM),
        ],
        out_specs=pl.BlockSpec((row_block, 1), lambda i: (i, 0)),
        out_shape=jax.ShapeDtypeStruct((b, 1), jnp.float32),
    )(y, w2d, bias2d)


def _round_up(x: int, m: int) -> int:
    return (x + m - 1) // m * m


def kernel(YhatFull, weights, bias, nodeOrder):
    b, n = YhatFull.shape
    v = nodeOrder.shape[0]

    n_pad = _round_up(n, 32 * _LANES)
    v_pad = _round_up(v, _LANES)
    extra = v_pad - v
    idx_pad = jnp.concatenate(
        [nodeOrder.astype(jnp.int32), n + jnp.arange(extra, dtype=jnp.int32)]
    )
    w_pad = jnp.concatenate([weights, jnp.zeros((extra,), jnp.float32)])

    w_full = _make_sc_scatter(n_pad, v_pad)(idx_pad, w_pad)
    w2d = w_full[:n].reshape(1, n)

    return _tc_matvec(YhatFull, w2d, bias.reshape(1, 1), 256)
